# baseline (speedup 1.0000x reference)
import dataclasses
import functools

import jax
import jax.numpy as jnp
import numpy as np
from jax import lax
from jax.experimental import pallas as pl
from jax.experimental.pallas import tpu as pltpu
from jax.experimental.pallas import tpu_sc as plsc

N = 10000
D = 128
C = 128
E = 320000

NC = 2
NS = 16
NW = NC * NS
L = 16

CHUNK = 64
NCH0 = 201
NCH1 = 114
MAXCH = max(NCH0, NCH1)
MAXR = MAXCH + 5
NSLOT = 3
NIRING = 8
N_ACC = 10240
ZPT = N_ACC // NS
A_PAD = 10016

BLK = 400
GRID = N // BLK


def _sc_compiler_params():
    cp = pltpu.CompilerParams()
    if "needs_layout_passes" in pltpu.CompilerParams.__dataclass_fields__:
        cp = dataclasses.replace(cp, needs_layout_passes=False)
    return cp


def _proj_body(x_ref, w_ref, asr_ref, adr_ref, h_ref, a1_ref, a2_ref):
    h = jnp.dot(x_ref[...], w_ref[...], preferred_element_type=jnp.float32)
    h_ref[...] = h
    a1_ref[...] = jnp.sum(h * asr_ref[...], axis=1, keepdims=True)
    a2_ref[...] = jnp.sum(h * adr_ref[...], axis=1, keepdims=True)


def _proj(x, w, a_src, a_dst):
    return pl.pallas_call(
        _proj_body,
        grid=(GRID,),
        in_specs=[
            pl.BlockSpec((BLK, D), lambda i: (i, 0)),
            pl.BlockSpec((D, C), lambda i: (0, 0)),
            pl.BlockSpec((1, C), lambda i: (0, 0)),
            pl.BlockSpec((1, C), lambda i: (0, 0)),
        ],
        out_specs=[
            pl.BlockSpec((BLK, C), lambda i: (i, 0)),
            pl.BlockSpec((BLK, 1), lambda i: (i, 0)),
            pl.BlockSpec((BLK, 1), lambda i: (i, 0)),
        ],
        out_shape=[
            jax.ShapeDtypeStruct((N, C), jnp.float32),
            jax.ShapeDtypeStruct((N, 1), jnp.float32),
            jax.ShapeDtypeStruct((N, 1), jnp.float32),
        ],
    )(x, w, a_src, a_dst)


def _finish(acc_ref, den_ref, b_ref):
    acc = acc_ref[0] + acc_ref[1]
    den = jnp.sum(den_ref[...], axis=0)
    o = acc / (den + 1e-16) + b_ref[...]
    return jnp.where(o > 0.0, o, jnp.exp(o) - 1.0)


def _finproj_body(acc_ref, den_ref, b_ref, w_ref, asr_ref, adr_ref,
                  h_ref, a1_ref, a2_ref):
    hin = _finish(acc_ref, den_ref, b_ref)
    h = jnp.dot(hin, w_ref[...], preferred_element_type=jnp.float32)
    h_ref[...] = h
    a1_ref[...] = jnp.sum(h * asr_ref[...], axis=1, keepdims=True)
    a2_ref[...] = jnp.sum(h * adr_ref[...], axis=1, keepdims=True)


def _finproj(acc, den, b, w, a_src, a_dst):
    return pl.pallas_call(
        _finproj_body,
        grid=(GRID,),
        in_specs=[
            pl.BlockSpec((NC, BLK, C), lambda i: (0, i, 0)),
            pl.BlockSpec((NW, BLK, 1), lambda i: (0, i, 0)),
            pl.BlockSpec((1, C), lambda i: (0, 0)),
            pl.BlockSpec((D, C), lambda i: (0, 0)),
            pl.BlockSpec((1, C), lambda i: (0, 0)),
            pl.BlockSpec((1, C), lambda i: (0, 0)),
        ],
        out_specs=[
            pl.BlockSpec((BLK, C), lambda i: (i, 0)),
            pl.BlockSpec((BLK, 1), lambda i: (i, 0)),
            pl.BlockSpec((BLK, 1), lambda i: (i, 0)),
        ],
        out_shape=[
            jax.ShapeDtypeStruct((N, C), jnp.float32),
            jax.ShapeDtypeStruct((N, 1), jnp.float32),
            jax.ShapeDtypeStruct((N, 1), jnp.float32),
        ],
    )(acc, den, b, w, a_src, a_dst)


def _final_body(acc_ref, den_ref, b_ref, o_ref):
    o_ref[...] = _finish(acc_ref, den_ref, b_ref)


def _final(acc, den, b):
    return pl.pallas_call(
        _final_body,
        grid=(GRID,),
        in_specs=[
            pl.BlockSpec((NC, BLK, C), lambda i: (0, i, 0)),
            pl.BlockSpec((NW, BLK, 1), lambda i: (0, i, 0)),
            pl.BlockSpec((1, C), lambda i: (0, 0)),
        ],
        out_specs=pl.BlockSpec((BLK, C), lambda i: (i, 0)),
        out_shape=jax.ShapeDtypeStruct((N, C), jnp.float32),
    )(acc, den, b)


_MESH = plsc.VectorSubcoreMesh(core_axis_name="core", subcore_axis_name="subcore")


@functools.partial(
    pl.kernel,
    out_type=[
        jax.ShapeDtypeStruct((NC * N_ACC, C), jnp.float32),
        jax.ShapeDtypeStruct((NW * N_ACC,), jnp.float32),
    ],
    mesh=_MESH,
    compiler_params=_sc_compiler_params(),
    scratch_types=[
        pltpu.VMEM((NIRING, CHUNK), jnp.int32),
        pltpu.VMEM((NIRING, CHUNK), jnp.int32),
        pltpu.VMEM((NSLOT, 3, CHUNK), jnp.float32),
        [pltpu.VMEM((CHUNK, C), jnp.float32)] * NSLOT,
        pltpu.VMEM((N_ACC,), jnp.float32),
        pltpu.VMEM_SHARED((N_ACC, C), jnp.float32),
        [pltpu.SemaphoreType.DMA] * NSLOT,
        [pltpu.SemaphoreType.DMA] * NSLOT,
        pltpu.SemaphoreType.DMA,
        pltpu.SemaphoreType.DMA,
    ],
)
def _sc_aggregate(h_hbm, asrc_hbm, adst_hbm, sidx_hbm, didx_hbm,
                  zrows_hbm, acc_out, den_out,
                  sidx_r, didx_r, small_v, rows, den_l,
                  acc_s, gsem, ssem, isem, jsem):
    asv = [small_v.at[s, 0] for s in range(NSLOT)]
    adv = [small_v.at[s, 1] for s in range(NSLOT)]
    wbuf = [small_v.at[s, 2] for s in range(NSLOT)]
    cid = lax.axis_index("core")
    sid = lax.axis_index("subcore")
    wid = sid * NC + cid
    zbase = sid * ZPT

    pltpu.sync_copy(zrows_hbm, acc_s.at[pl.ds(zbase, ZPT)])

    @pl.loop(0, N_ACC, step=L)
    def _zd(g):
        den_l.at[pl.ds(g, L)][...] = jnp.zeros((L,), jnp.float32)

    ncb = jnp.where(cid == 0, NCH0, NCH1)

    for r in range(4):
        pltpu.sync_copy(sidx_hbm.at[wid, r], sidx_r.at[r])
        pltpu.sync_copy(didx_hbm.at[wid, r], didx_r.at[r])

    plsc.subcore_barrier()

    def sidx_copy(r):
        return pltpu.make_async_copy(sidx_hbm.at[wid, r],
                                     sidx_r.at[r & (NIRING - 1)], isem)

    def didx_copy(r):
        return pltpu.make_async_copy(didx_hbm.at[wid, r],
                                     didx_r.at[r & (NIRING - 1)], jsem)

    def didx(j):
        return didx_r.at[j & (NIRING - 1)]

    def gather_copies(j, s):
        sr = sidx_r.at[j & (NIRING - 1)]
        return (pltpu.make_async_copy(h_hbm.at[sr], rows[s], gsem[s]),
                pltpu.make_async_copy(asrc_hbm.at[sr], asv[s], gsem[s]),
                pltpu.make_async_copy(adst_hbm.at[didx(j)], adv[s], gsem[s]))

    def start_gathers(j, s):
        for cp in gather_copies(j, s):
            cp.start()

    def wait_gathers(j, s):
        for cp in gather_copies(j, s):
            cp.wait()

    def start_scatters(j, s):
        pltpu.async_copy(rows[s], acc_s.at[didx(j)], ssem[s], add=True)

    def wait_scatters(j, s):
        pltpu.make_async_copy(rows[s], acc_s.at[didx(j)], ssem[s]).wait()

    def process(j, s):
        wait_gathers(j, s)
        for k in range(CHUNK // L):
            sl = pl.ds(L * k, L)
            e = asv[s].at[sl][...] + adv[s].at[sl][...]
            e = jnp.where(e > 0.0, e, 0.2 * e)
            w = jnp.exp(e)
            wbuf[s].at[sl][...] = w
            di = didx_r.at[j & (NIRING - 1), sl][...]
            plsc.addupdate_scatter(den_l, [di], w)

        @pl.loop(0, CHUNK, unroll=2)
        def _scale(e):
            ee = jnp.full((L,), e, jnp.int32)
            wsp = plsc.load_gather(wbuf[s], [ee])
            for b in range(C // L):
                sl = pl.ds(L * b, L)
                rows[s].at[e, sl][...] = rows[s].at[e, sl][...] * wsp

        start_scatters(j, s)

    start_gathers(0, 0)
    start_gathers(1, 1)
    process(0, 0)
    sidx_copy(4).start()
    didx_copy(4).start()
    start_gathers(2, 2)
    process(1, 1)
    sidx_copy(5).start()
    didx_copy(5).start()
    wait_scatters(0, 0)
    start_gathers(3, 0)
    process(2, 2)
    sidx_copy(6).start()
    didx_copy(6).start()

    @pl.loop(3, ncb, step=NSLOT)
    def _main(m):
        for t in range(NSLOT):
            j = m + t
            sn = (t + 1) % NSLOT
            wait_scatters(j - 2, sn)
            sidx_copy(j + 1).wait()
            didx_copy(j + 1).wait()
            start_gathers(j + 1, sn)
            process(j, t)
            sidx_copy(j + 4).start()
            didx_copy(j + 4).start()

    wait_scatters(ncb - 2, 1)
    wait_scatters(ncb - 1, 2)
    wait_gathers(ncb, 0)
    for k in range(1, 4):
        sidx_copy(ncb + k).wait()
        didx_copy(ncb + k).wait()

    pltpu.sync_copy(den_l, den_out.at[pl.ds(wid * N_ACC, N_ACC)])

    plsc.subcore_barrier()

    obase = cid * N_ACC + zbase
    pltpu.sync_copy(acc_s.at[pl.ds(zbase, ZPT)], acc_out.at[pl.ds(obase, ZPT)])


def _layer_aggregate(h, asrc, adst, sidx3, didx3, zrows):
    asrc_p = jnp.pad(asrc[:, 0], (0, A_PAD - N))
    adst_p = jnp.pad(adst[:, 0], (0, A_PAD - N))
    acc, den = _sc_aggregate(h, asrc_p, adst_p, sidx3, didx3, zrows)
    acc = acc.reshape(NC, N_ACC, C)
    den = den.reshape(NW, N_ACC, 1)
    return acc, den


def kernel(x, edge_index, W1, a_src1, a_dst1, b1, W2, a_src2, a_dst2, b2):
    src = edge_index[0]
    dst = edge_index[1]
    counts = [(NCH0 if w % NC == 0 else NCH1) * CHUNK for w in range(NW)]
    total = sum(counts)
    src_p = jnp.concatenate([src, jnp.zeros((total - E,), jnp.int32)])
    dst_p = jnp.concatenate([dst, jnp.full((total - E,), N, jnp.int32)])
    stiles, dtiles = [], []
    off = 0
    for w in range(NW):
        n = counts[w]
        stiles.append(jnp.pad(src_p[off:off + n], (0, MAXR * CHUNK - n)))
        dtiles.append(jnp.pad(dst_p[off:off + n], (0, MAXR * CHUNK - n),
                              constant_values=N))
        off += n
    sidx3 = jnp.stack(stiles).reshape(NW, MAXR, CHUNK)
    didx3 = jnp.stack(dtiles).reshape(NW, MAXR, CHUNK)
    zrows = jnp.zeros((ZPT, C), jnp.float32)

    b1r = b1.reshape(1, C)
    b2r = b2.reshape(1, C)

    h1, as1, ad1 = _proj(x, W1, a_src1, a_dst1)
    acc1, den1 = _layer_aggregate(h1, as1, ad1, sidx3, didx3, zrows)
    h2, as2, ad2 = _finproj(acc1, den1, b1r, W2, a_src2, a_dst2)
    acc2, den2 = _layer_aggregate(h2, as2, ad2, sidx3, didx3, zrows)
    return _final(acc2, den2, b2r)


# speedup vs baseline: 1.4373x; 1.4373x over previous
import dataclasses
import functools

import jax
import jax.numpy as jnp
import numpy as np
from jax import lax
from jax.experimental import pallas as pl
from jax.experimental.pallas import tpu as pltpu
from jax.experimental.pallas import tpu_sc as plsc

N = 10000
D = 128
C = 128
E = 320000

NC = 2
NS = 16
NW = NC * NS
L = 16

CHUNK = 64
NCH0 = 201
NCH1 = 114
MAXCH = max(NCH0, NCH1)
MAXR = MAXCH + 5
NSLOT = 3
NIRING = 8
N_ACC = 10240
ZPT = N_ACC // NS
A_PAD = 10016

BLK = 400
GRID = N // BLK


def _sc_compiler_params():
    cp = pltpu.CompilerParams()
    if "needs_layout_passes" in pltpu.CompilerParams.__dataclass_fields__:
        cp = dataclasses.replace(cp, needs_layout_passes=False)
    return cp


def _proj_body(x_ref, w_ref, asr_ref, adr_ref, h_ref, a1_ref, a2_ref):
    h = jnp.dot(x_ref[...], w_ref[...], preferred_element_type=jnp.float32)
    h_ref[...] = h
    a1_ref[...] = jnp.sum(h * asr_ref[...], axis=1, keepdims=True)
    a2_ref[...] = jnp.sum(h * adr_ref[...], axis=1, keepdims=True)


def _proj(x, w, a_src, a_dst):
    return pl.pallas_call(
        _proj_body,
        grid=(GRID,),
        in_specs=[
            pl.BlockSpec((BLK, D), lambda i: (i, 0)),
            pl.BlockSpec((D, C), lambda i: (0, 0)),
            pl.BlockSpec((1, C), lambda i: (0, 0)),
            pl.BlockSpec((1, C), lambda i: (0, 0)),
        ],
        out_specs=[
            pl.BlockSpec((BLK, C), lambda i: (i, 0)),
            pl.BlockSpec((BLK, 1), lambda i: (i, 0)),
            pl.BlockSpec((BLK, 1), lambda i: (i, 0)),
        ],
        out_shape=[
            jax.ShapeDtypeStruct((N, C), jnp.float32),
            jax.ShapeDtypeStruct((N, 1), jnp.float32),
            jax.ShapeDtypeStruct((N, 1), jnp.float32),
        ],
    )(x, w, a_src, a_dst)


def _finish(acc_ref, den_ref, b_ref):
    acc = acc_ref[0] + acc_ref[1]
    den = den_ref[0] + den_ref[1]
    o = acc / (den + 1e-16) + b_ref[...]
    return jnp.where(o > 0.0, o, jnp.exp(o) - 1.0)


def _finproj_body(acc_ref, den_ref, b_ref, w_ref, asr_ref, adr_ref,
                  h_ref, a1_ref, a2_ref):
    hin = _finish(acc_ref, den_ref, b_ref)
    h = jnp.dot(hin, w_ref[...], preferred_element_type=jnp.float32)
    h_ref[...] = h
    a1_ref[...] = jnp.sum(h * asr_ref[...], axis=1, keepdims=True)
    a2_ref[...] = jnp.sum(h * adr_ref[...], axis=1, keepdims=True)


def _finproj(acc, den, b, w, a_src, a_dst):
    return pl.pallas_call(
        _finproj_body,
        grid=(GRID,),
        in_specs=[
            pl.BlockSpec((NC, BLK, C), lambda i: (0, i, 0)),
            pl.BlockSpec((NC, BLK, 1), lambda i: (0, i, 0)),
            pl.BlockSpec((1, C), lambda i: (0, 0)),
            pl.BlockSpec((D, C), lambda i: (0, 0)),
            pl.BlockSpec((1, C), lambda i: (0, 0)),
            pl.BlockSpec((1, C), lambda i: (0, 0)),
        ],
        out_specs=[
            pl.BlockSpec((BLK, C), lambda i: (i, 0)),
            pl.BlockSpec((BLK, 1), lambda i: (i, 0)),
            pl.BlockSpec((BLK, 1), lambda i: (i, 0)),
        ],
        out_shape=[
            jax.ShapeDtypeStruct((N, C), jnp.float32),
            jax.ShapeDtypeStruct((N, 1), jnp.float32),
            jax.ShapeDtypeStruct((N, 1), jnp.float32),
        ],
    )(acc, den, b, w, a_src, a_dst)


def _final_body(acc_ref, den_ref, b_ref, o_ref):
    o_ref[...] = _finish(acc_ref, den_ref, b_ref)


def _final(acc, den, b):
    return pl.pallas_call(
        _final_body,
        grid=(GRID,),
        in_specs=[
            pl.BlockSpec((NC, BLK, C), lambda i: (0, i, 0)),
            pl.BlockSpec((NC, BLK, 1), lambda i: (0, i, 0)),
            pl.BlockSpec((1, C), lambda i: (0, 0)),
        ],
        out_specs=pl.BlockSpec((BLK, C), lambda i: (i, 0)),
        out_shape=jax.ShapeDtypeStruct((N, C), jnp.float32),
    )(acc, den, b)


_MESH = plsc.VectorSubcoreMesh(core_axis_name="core", subcore_axis_name="subcore")


@functools.partial(
    pl.kernel,
    out_type=[
        jax.ShapeDtypeStruct((NC * N_ACC, C), jnp.float32),
        jax.ShapeDtypeStruct((NC * N_ACC,), jnp.float32),
    ],
    mesh=_MESH,
    compiler_params=_sc_compiler_params(),
    scratch_types=[
        pltpu.VMEM((NIRING, CHUNK), jnp.int32),
        pltpu.VMEM((NIRING, CHUNK), jnp.int32),
        pltpu.VMEM((NSLOT, 3, CHUNK), jnp.float32),
        [pltpu.VMEM((CHUNK, C), jnp.float32)] * NSLOT,
        pltpu.VMEM_SHARED((N_ACC, C), jnp.float32),
        pltpu.VMEM_SHARED((N_ACC,), jnp.float32),
        [pltpu.SemaphoreType.DMA] * NSLOT,
        [pltpu.SemaphoreType.DMA] * NSLOT,
        pltpu.SemaphoreType.DMA,
        pltpu.SemaphoreType.DMA,
    ],
)
def _sc_aggregate(h_hbm, asrc_hbm, adst_hbm, sidx_hbm, didx_hbm,
                  zrows_hbm, zden_hbm, acc_out, den_out,
                  sidx_r, didx_r, small_v, rows,
                  acc_s, den_s, gsem, ssem, isem, jsem):
    asv = [small_v.at[s, 0] for s in range(NSLOT)]
    adv = [small_v.at[s, 1] for s in range(NSLOT)]
    wbuf = [small_v.at[s, 2] for s in range(NSLOT)]
    cid = lax.axis_index("core")
    sid = lax.axis_index("subcore")
    wid = sid * NC + cid
    zbase = sid * ZPT

    pltpu.sync_copy(zrows_hbm, acc_s.at[pl.ds(zbase, ZPT)])
    pltpu.sync_copy(zden_hbm, den_s.at[pl.ds(zbase, ZPT)])

    ncb = jnp.where(cid == 0, NCH0, NCH1)

    for r in range(4):
        pltpu.sync_copy(sidx_hbm.at[wid, r], sidx_r.at[r])
        pltpu.sync_copy(didx_hbm.at[wid, r], didx_r.at[r])

    plsc.subcore_barrier()

    def sidx_copy(r):
        return pltpu.make_async_copy(sidx_hbm.at[wid, r],
                                     sidx_r.at[r & (NIRING - 1)], isem)

    def didx_copy(r):
        return pltpu.make_async_copy(didx_hbm.at[wid, r],
                                     didx_r.at[r & (NIRING - 1)], jsem)

    def didx(j):
        return didx_r.at[j & (NIRING - 1)]

    def gather_copies(j, s):
        sr = sidx_r.at[j & (NIRING - 1)]
        return (pltpu.make_async_copy(h_hbm.at[sr], rows[s], gsem[s]),
                pltpu.make_async_copy(asrc_hbm.at[sr], asv[s], gsem[s]),
                pltpu.make_async_copy(adst_hbm.at[didx(j)], adv[s], gsem[s]))

    def start_gathers(j, s):
        for cp in gather_copies(j, s):
            cp.start()

    def wait_gathers(j, s):
        for cp in gather_copies(j, s):
            cp.wait()

    def start_scatters(j, s):
        pltpu.async_copy(wbuf[s], den_s.at[didx(j)], ssem[s], add=True)
        pltpu.async_copy(rows[s], acc_s.at[didx(j)], ssem[s], add=True)

    def wait_scatters(j, s):
        pltpu.make_async_copy(wbuf[s], den_s.at[didx(j)], ssem[s]).wait()
        pltpu.make_async_copy(rows[s], acc_s.at[didx(j)], ssem[s]).wait()

    def process(j, s):
        wait_gathers(j, s)
        for k in range(CHUNK // L):
            sl = pl.ds(L * k, L)
            e = asv[s].at[sl][...] + adv[s].at[sl][...]
            e = jnp.where(e > 0.0, e, 0.2 * e)
            wbuf[s].at[sl][...] = jnp.exp(e)

        @pl.loop(0, CHUNK, unroll=2)
        def _scale(e):
            ee = jnp.full((L,), e, jnp.int32)
            wsp = plsc.load_gather(wbuf[s], [ee])
            for b in range(C // L):
                sl = pl.ds(L * b, L)
                rows[s].at[e, sl][...] = rows[s].at[e, sl][...] * wsp

        start_scatters(j, s)

    start_gathers(0, 0)
    start_gathers(1, 1)
    process(0, 0)
    sidx_copy(4).start()
    didx_copy(4).start()
    start_gathers(2, 2)
    process(1, 1)
    sidx_copy(5).start()
    didx_copy(5).start()
    wait_scatters(0, 0)
    start_gathers(3, 0)
    process(2, 2)
    sidx_copy(6).start()
    didx_copy(6).start()

    @pl.loop(3, ncb, step=NSLOT)
    def _main(m):
        for t in range(NSLOT):
            j = m + t
            sn = (t + 1) % NSLOT
            wait_scatters(j - 2, sn)
            sidx_copy(j + 1).wait()
            didx_copy(j + 1).wait()
            start_gathers(j + 1, sn)
            process(j, t)
            sidx_copy(j + 4).start()
            didx_copy(j + 4).start()

    wait_scatters(ncb - 2, 1)
    wait_scatters(ncb - 1, 2)
    wait_gathers(ncb, 0)
    for k in range(1, 4):
        sidx_copy(ncb + k).wait()
        didx_copy(ncb + k).wait()

    plsc.subcore_barrier()

    obase = cid * N_ACC + zbase
    pltpu.sync_copy(acc_s.at[pl.ds(zbase, ZPT)], acc_out.at[pl.ds(obase, ZPT)])
    pltpu.sync_copy(den_s.at[pl.ds(zbase, ZPT)], den_out.at[pl.ds(obase, ZPT)])


def _layer_aggregate(h, asrc, adst, sidx3, didx3, zrows, zden):
    asrc_p = jnp.pad(asrc[:, 0], (0, A_PAD - N))
    adst_p = jnp.pad(adst[:, 0], (0, A_PAD - N))
    acc, den = _sc_aggregate(h, asrc_p, adst_p, sidx3, didx3, zrows, zden)
    acc = acc.reshape(NC, N_ACC, C)
    den = den.reshape(NC, N_ACC, 1)
    return acc, den


def kernel(x, edge_index, W1, a_src1, a_dst1, b1, W2, a_src2, a_dst2, b2):
    src = edge_index[0]
    dst = edge_index[1]
    counts = [(NCH0 if w % NC == 0 else NCH1) * CHUNK for w in range(NW)]
    total = sum(counts)
    src_p = jnp.concatenate([src, jnp.zeros((total - E,), jnp.int32)])
    dst_p = jnp.concatenate([dst, jnp.full((total - E,), N, jnp.int32)])
    stiles, dtiles = [], []
    off = 0
    for w in range(NW):
        n = counts[w]
        stiles.append(jnp.pad(src_p[off:off + n], (0, MAXR * CHUNK - n)))
        dtiles.append(jnp.pad(dst_p[off:off + n], (0, MAXR * CHUNK - n),
                              constant_values=N))
        off += n
    sidx3 = jnp.stack(stiles).reshape(NW, MAXR, CHUNK)
    didx3 = jnp.stack(dtiles).reshape(NW, MAXR, CHUNK)
    zrows = jnp.zeros((ZPT, C), jnp.float32)
    zden = jnp.zeros((ZPT,), jnp.float32)

    b1r = b1.reshape(1, C)
    b2r = b2.reshape(1, C)

    h1, as1, ad1 = _proj(x, W1, a_src1, a_dst1)
    acc1, den1 = _layer_aggregate(h1, as1, ad1, sidx3, didx3, zrows, zden)
    h2, as2, ad2 = _finproj(acc1, den1, b1r, W2, a_src2, a_dst2)
    acc2, den2 = _layer_aggregate(h2, as2, ad2, sidx3, didx3, zrows, zden)
    return _final(acc2, den2, b2r)


# speedup vs baseline: 1.4398x; 1.0017x over previous
import dataclasses
import functools

import jax
import jax.numpy as jnp
import numpy as np
from jax import lax
from jax.experimental import pallas as pl
from jax.experimental.pallas import tpu as pltpu
from jax.experimental.pallas import tpu_sc as plsc

N = 10000
D = 128
C = 128
E = 320000

NC = 2
NS = 16
NW = NC * NS
L = 16

CHUNK = 64
NCH0 = 207
NCH1 = 108
MAXCH = max(NCH0, NCH1)
MAXR = MAXCH + 5
NSLOT = 3
NIRING = 8
N_ACC = 10240
ZPT = N_ACC // NS
A_PAD = 10016

BLK = 400
GRID = N // BLK


def _sc_compiler_params():
    cp = pltpu.CompilerParams()
    if "needs_layout_passes" in pltpu.CompilerParams.__dataclass_fields__:
        cp = dataclasses.replace(cp, needs_layout_passes=False)
    return cp


def _proj_body(x_ref, w_ref, asr_ref, adr_ref, h_ref, a1_ref, a2_ref):
    h = jnp.dot(x_ref[...], w_ref[...], preferred_element_type=jnp.float32)
    h_ref[...] = h
    a1_ref[...] = jnp.sum(h * asr_ref[...], axis=1, keepdims=True)
    a2_ref[...] = jnp.sum(h * adr_ref[...], axis=1, keepdims=True)


def _proj(x, w, a_src, a_dst):
    return pl.pallas_call(
        _proj_body,
        grid=(GRID,),
        in_specs=[
            pl.BlockSpec((BLK, D), lambda i: (i, 0)),
            pl.BlockSpec((D, C), lambda i: (0, 0)),
            pl.BlockSpec((1, C), lambda i: (0, 0)),
            pl.BlockSpec((1, C), lambda i: (0, 0)),
        ],
        out_specs=[
            pl.BlockSpec((BLK, C), lambda i: (i, 0)),
            pl.BlockSpec((BLK, 1), lambda i: (i, 0)),
            pl.BlockSpec((BLK, 1), lambda i: (i, 0)),
        ],
        out_shape=[
            jax.ShapeDtypeStruct((N, C), jnp.float32),
            jax.ShapeDtypeStruct((N, 1), jnp.float32),
            jax.ShapeDtypeStruct((N, 1), jnp.float32),
        ],
    )(x, w, a_src, a_dst)


def _finish(acc_ref, den_ref, b_ref):
    acc = acc_ref[0] + acc_ref[1]
    den = den_ref[0] + den_ref[1]
    o = acc / (den + 1e-16) + b_ref[...]
    return jnp.where(o > 0.0, o, jnp.exp(o) - 1.0)


def _finproj_body(acc_ref, den_ref, b_ref, w_ref, asr_ref, adr_ref,
                  h_ref, a1_ref, a2_ref):
    hin = _finish(acc_ref, den_ref, b_ref)
    h = jnp.dot(hin, w_ref[...], preferred_element_type=jnp.float32)
    h_ref[...] = h
    a1_ref[...] = jnp.sum(h * asr_ref[...], axis=1, keepdims=True)
    a2_ref[...] = jnp.sum(h * adr_ref[...], axis=1, keepdims=True)


def _finproj(acc, den, b, w, a_src, a_dst):
    return pl.pallas_call(
        _finproj_body,
        grid=(GRID,),
        in_specs=[
            pl.BlockSpec((NC, BLK, C), lambda i: (0, i, 0)),
            pl.BlockSpec((NC, BLK, 1), lambda i: (0, i, 0)),
            pl.BlockSpec((1, C), lambda i: (0, 0)),
            pl.BlockSpec((D, C), lambda i: (0, 0)),
            pl.BlockSpec((1, C), lambda i: (0, 0)),
            pl.BlockSpec((1, C), lambda i: (0, 0)),
        ],
        out_specs=[
            pl.BlockSpec((BLK, C), lambda i: (i, 0)),
            pl.BlockSpec((BLK, 1), lambda i: (i, 0)),
            pl.BlockSpec((BLK, 1), lambda i: (i, 0)),
        ],
        out_shape=[
            jax.ShapeDtypeStruct((N, C), jnp.float32),
            jax.ShapeDtypeStruct((N, 1), jnp.float32),
            jax.ShapeDtypeStruct((N, 1), jnp.float32),
        ],
    )(acc, den, b, w, a_src, a_dst)


def _final_body(acc_ref, den_ref, b_ref, o_ref):
    o_ref[...] = _finish(acc_ref, den_ref, b_ref)


def _final(acc, den, b):
    return pl.pallas_call(
        _final_body,
        grid=(GRID,),
        in_specs=[
            pl.BlockSpec((NC, BLK, C), lambda i: (0, i, 0)),
            pl.BlockSpec((NC, BLK, 1), lambda i: (0, i, 0)),
            pl.BlockSpec((1, C), lambda i: (0, 0)),
        ],
        out_specs=pl.BlockSpec((BLK, C), lambda i: (i, 0)),
        out_shape=jax.ShapeDtypeStruct((N, C), jnp.float32),
    )(acc, den, b)


_MESH = plsc.VectorSubcoreMesh(core_axis_name="core", subcore_axis_name="subcore")


@functools.partial(
    pl.kernel,
    out_type=[
        jax.ShapeDtypeStruct((NC * N_ACC, C), jnp.float32),
        jax.ShapeDtypeStruct((NC * N_ACC,), jnp.float32),
    ],
    mesh=_MESH,
    compiler_params=_sc_compiler_params(),
    scratch_types=[
        pltpu.VMEM((NIRING, CHUNK), jnp.int32),
        pltpu.VMEM((NIRING, CHUNK), jnp.int32),
        pltpu.VMEM((NSLOT, 3, CHUNK), jnp.float32),
        [pltpu.VMEM((CHUNK, C), jnp.float32)] * NSLOT,
        pltpu.VMEM_SHARED((N_ACC, C), jnp.float32),
        pltpu.VMEM_SHARED((N_ACC,), jnp.float32),
        [pltpu.SemaphoreType.DMA] * NSLOT,
        [pltpu.SemaphoreType.DMA] * NSLOT,
        pltpu.SemaphoreType.DMA,
        pltpu.SemaphoreType.DMA,
    ],
)
def _sc_aggregate(h_hbm, asrc_hbm, adst_hbm, sidx_hbm, didx_hbm,
                  zrows_hbm, zden_hbm, acc_out, den_out,
                  sidx_r, didx_r, small_v, rows,
                  acc_s, den_s, gsem, ssem, isem, jsem):
    asv = [small_v.at[s, 0] for s in range(NSLOT)]
    adv = [small_v.at[s, 1] for s in range(NSLOT)]
    wbuf = [small_v.at[s, 2] for s in range(NSLOT)]
    cid = lax.axis_index("core")
    sid = lax.axis_index("subcore")
    wid = sid * NC + cid
    zbase = sid * ZPT

    pltpu.sync_copy(zrows_hbm, acc_s.at[pl.ds(zbase, ZPT)])
    pltpu.sync_copy(zden_hbm, den_s.at[pl.ds(zbase, ZPT)])

    ncb = jnp.where(cid == 0, NCH0, NCH1)

    for r in range(4):
        pltpu.sync_copy(sidx_hbm.at[wid, r], sidx_r.at[r])
        pltpu.sync_copy(didx_hbm.at[wid, r], didx_r.at[r])

    plsc.subcore_barrier()

    def sidx_copy(r):
        return pltpu.make_async_copy(sidx_hbm.at[wid, r],
                                     sidx_r.at[r & (NIRING - 1)], isem)

    def didx_copy(r):
        return pltpu.make_async_copy(didx_hbm.at[wid, r],
                                     didx_r.at[r & (NIRING - 1)], jsem)

    def didx(j):
        return didx_r.at[j & (NIRING - 1)]

    def gather_copies(j, s):
        sr = sidx_r.at[j & (NIRING - 1)]
        return (pltpu.make_async_copy(h_hbm.at[sr], rows[s], gsem[s]),
                pltpu.make_async_copy(asrc_hbm.at[sr], asv[s], gsem[s]),
                pltpu.make_async_copy(adst_hbm.at[didx(j)], adv[s], gsem[s]))

    def start_gathers(j, s):
        for cp in gather_copies(j, s):
            cp.start()

    def wait_gathers(j, s):
        for cp in gather_copies(j, s):
            cp.wait()

    def start_scatters(j, s):
        pltpu.async_copy(wbuf[s], den_s.at[didx(j)], ssem[s], add=True)
        pltpu.async_copy(rows[s], acc_s.at[didx(j)], ssem[s], add=True)

    def wait_scatters(j, s):
        pltpu.make_async_copy(wbuf[s], den_s.at[didx(j)], ssem[s]).wait()
        pltpu.make_async_copy(rows[s], acc_s.at[didx(j)], ssem[s]).wait()

    def process(j, s):
        wait_gathers(j, s)
        for k in range(CHUNK // L):
            sl = pl.ds(L * k, L)
            e = asv[s].at[sl][...] + adv[s].at[sl][...]
            e = jnp.where(e > 0.0, e, 0.2 * e)
            wbuf[s].at[sl][...] = jnp.exp(e)

        @pl.loop(0, CHUNK, unroll=2)
        def _scale(e):
            ee = jnp.full((L,), e, jnp.int32)
            wsp = plsc.load_gather(wbuf[s], [ee])
            for b in range(C // L):
                sl = pl.ds(L * b, L)
                rows[s].at[e, sl][...] = rows[s].at[e, sl][...] * wsp

        start_scatters(j, s)

    start_gathers(0, 0)
    start_gathers(1, 1)
    process(0, 0)
    sidx_copy(4).start()
    didx_copy(4).start()
    start_gathers(2, 2)
    process(1, 1)
    sidx_copy(5).start()
    didx_copy(5).start()
    wait_scatters(0, 0)
    start_gathers(3, 0)
    process(2, 2)
    sidx_copy(6).start()
    didx_copy(6).start()

    @pl.loop(3, ncb, step=NSLOT)
    def _main(m):
        for t in range(NSLOT):
            j = m + t
            sn = (t + 1) % NSLOT
            wait_scatters(j - 2, sn)
            sidx_copy(j + 1).wait()
            didx_copy(j + 1).wait()
            start_gathers(j + 1, sn)
            process(j, t)
            sidx_copy(j + 4).start()
            didx_copy(j + 4).start()

    wait_scatters(ncb - 2, 1)
    wait_scatters(ncb - 1, 2)
    wait_gathers(ncb, 0)
    for k in range(1, 4):
        sidx_copy(ncb + k).wait()
        didx_copy(ncb + k).wait()

    plsc.subcore_barrier()

    obase = cid * N_ACC + zbase
    pltpu.sync_copy(acc_s.at[pl.ds(zbase, ZPT)], acc_out.at[pl.ds(obase, ZPT)])
    pltpu.sync_copy(den_s.at[pl.ds(zbase, ZPT)], den_out.at[pl.ds(obase, ZPT)])


def _layer_aggregate(h, asrc, adst, sidx3, didx3, zrows, zden):
    asrc_p = jnp.pad(asrc[:, 0], (0, A_PAD - N))
    adst_p = jnp.pad(adst[:, 0], (0, A_PAD - N))
    acc, den = _sc_aggregate(h, asrc_p, adst_p, sidx3, didx3, zrows, zden)
    acc = acc.reshape(NC, N_ACC, C)
    den = den.reshape(NC, N_ACC, 1)
    return acc, den


def kernel(x, edge_index, W1, a_src1, a_dst1, b1, W2, a_src2, a_dst2, b2):
    src = edge_index[0]
    dst = edge_index[1]
    counts = [(NCH0 if w % NC == 0 else NCH1) * CHUNK for w in range(NW)]
    total = sum(counts)
    src_p = jnp.concatenate([src, jnp.zeros((total - E,), jnp.int32)])
    dst_p = jnp.concatenate([dst, jnp.full((total - E,), N, jnp.int32)])
    stiles, dtiles = [], []
    off = 0
    for w in range(NW):
        n = counts[w]
        stiles.append(jnp.pad(src_p[off:off + n], (0, MAXR * CHUNK - n)))
        dtiles.append(jnp.pad(dst_p[off:off + n], (0, MAXR * CHUNK - n),
                              constant_values=N))
        off += n
    sidx3 = jnp.stack(stiles).reshape(NW, MAXR, CHUNK)
    didx3 = jnp.stack(dtiles).reshape(NW, MAXR, CHUNK)
    zrows = jnp.zeros((ZPT, C), jnp.float32)
    zden = jnp.zeros((ZPT,), jnp.float32)

    b1r = b1.reshape(1, C)
    b2r = b2.reshape(1, C)

    h1, as1, ad1 = _proj(x, W1, a_src1, a_dst1)
    acc1, den1 = _layer_aggregate(h1, as1, ad1, sidx3, didx3, zrows, zden)
    h2, as2, ad2 = _finproj(acc1, den1, b1r, W2, a_src2, a_dst2)
    acc2, den2 = _layer_aggregate(h2, as2, ad2, sidx3, didx3, zrows, zden)
    return _final(acc2, den2, b2r)
